# Initial kernel scaffold; baseline (speedup 1.0000x reference)
import jax
import jax.numpy as jnp
from jax.experimental import pallas as pl


def kernel(x, edge_index, feature_mtx_static, layers, inner_edges_0, inner_edges_1, inner_edges_2, forward_edges_0, forward_edges_1, forward_edges_2, backward_edges_0, backward_edges_1, backward_edges_2, batch_vec, W_up, b_up, W_in, b_in, W_fw, b_fw, W_bw, b_bw, W_lin, b_lin):
    raise NotImplementedError("write your pallas kernel here")


# speedup vs baseline: 4.5087x; 4.5087x over previous
import functools

import jax
import jax.numpy as jnp
from jax import lax
from jax.experimental import pallas as pl
from jax.experimental.pallas import tpu as pltpu
from jax.experimental.pallas import tpu_sc as plsc

N = 10000
DYN = 128
NSEG = 10240
ROWS_PER_SUB = NSEG // 16
CHUNK = 128
NW = 32
DUMP = N + 100

_mesh = plsc.VectorSubcoreMesh(core_axis_name="c", subcore_axis_name="s")


def _pad_edges(e, mult=NW * CHUNK):
    n = e.shape[1]
    npad = (-n) % mult
    src = jnp.concatenate([e[0], jnp.zeros((npad,), e.dtype)])
    dst = jnp.concatenate([e[1], jnp.full((npad,), DUMP, e.dtype)])
    return src.astype(jnp.int32), dst.astype(jnp.int32)


def _scatter_rows(g, src, dst, zeros, const_rows=False):
    e_pad = dst.shape[0]
    per_w = e_pad // NW
    nch = per_w // CHUNK

    @functools.partial(
        pl.kernel,
        mesh=_mesh,
        out_type=jax.ShapeDtypeStruct((2 * NSEG, DYN), jnp.float32),
        scratch_types=[
            pltpu.VMEM((CHUNK,), jnp.int32),
            pltpu.VMEM((CHUNK,), jnp.int32),
            pltpu.VMEM((CHUNK,), jnp.int32),
            pltpu.VMEM((CHUNK, DYN), jnp.float32),
            pltpu.VMEM((CHUNK, DYN), jnp.float32),
            pltpu.VMEM_SHARED((NSEG, DYN), jnp.float32),
            pltpu.SemaphoreType.DMA,
        ],
    )
    def k(g_hbm, src_hbm, dst_hbm, z_hbm, out_hbm, idx_s, idx_d, idx_b,
          rows, zb, acc, sem):
        cid = lax.axis_index("c")
        sid = lax.axis_index("s")
        wid = sid * 2 + cid
        lanes = lax.iota(jnp.int32, 16)

        def fill_idx(base):
            for kk in range(8):
                idx_b[pl.ds(kk * 16, 16)] = base + kk * 16 + lanes

        pltpu.sync_copy(z_hbm, zb)
        if const_rows:
            pltpu.sync_copy(g_hbm, rows)

        def zinit(j, carry):
            fill_idx(sid * ROWS_PER_SUB + j * CHUNK)
            pltpu.sync_copy(zb, acc.at[idx_b])
            return carry

        lax.fori_loop(0, ROWS_PER_SUB // CHUNK, zinit, 0)
        plsc.subcore_barrier()

        def body(j, carry):
            b = wid * per_w + j * CHUNK
            if not const_rows:
                pltpu.sync_copy(src_hbm.at[pl.ds(b, CHUNK)], idx_s)
            pltpu.sync_copy(dst_hbm.at[pl.ds(b, CHUNK)], idx_d)
            if not const_rows:
                pltpu.async_copy(g_hbm.at[idx_s], rows, sem).wait()
            pltpu.sync_copy(rows, acc.at[idx_d], add=True)
            return carry

        lax.fori_loop(0, nch, body, 0)
        plsc.subcore_barrier()

        def outb(j, carry):
            o = sid * ROWS_PER_SUB + j * CHUNK
            fill_idx(o)
            pltpu.sync_copy(acc.at[idx_b], zb)
            pltpu.sync_copy(zb, out_hbm.at[pl.ds(cid * NSEG + o, CHUNK)])
            return carry

        lax.fori_loop(0, ROWS_PER_SUB // CHUNK, outb, 0)

    if src is None:
        src = dst
    return k(g, src, dst, zeros)


def _col(P, c):
    e = (lax.broadcasted_iota(jnp.int32, (8, 1), 0) == c).astype(jnp.float32)
    return jnp.dot(P, e, preferred_element_type=jnp.float32)


def _prep_call(feature_mtx_static, W_in, W_fw, deg6):
    def body(st_ref, wi_ref, wf_ref, dg_ref, sin_ref, sfw_ref, dinv_ref):
        st = st_ref[...]
        sin_ref[...] = jnp.dot(st, wi_ref[DYN:, :],
                               preferred_element_type=jnp.float32)
        sfw_ref[...] = jnp.dot(st, wf_ref[DYN:, :],
                               preferred_element_type=jnp.float32)
        dinv_ref[...] = lax.rsqrt(1.0 + dg_ref[...])

    return pl.pallas_call(
        body,
        out_shape=(
            jax.ShapeDtypeStruct((N, DYN), jnp.float32),
            jax.ShapeDtypeStruct((N, DYN), jnp.float32),
            jax.ShapeDtypeStruct((N, 6), jnp.float32),
        ),
    )(feature_mtx_static, W_in, W_fw, deg6)


def _first_g(x, W_up, P):
    def body(x_ref, w_ref, p_ref, g_ref):
        g_ref[...] = jnp.dot(x_ref[...], w_ref[...],
                             preferred_element_type=jnp.float32) \
            * _col(p_ref[...], 0)

    return pl.pallas_call(
        body, out_shape=jax.ShapeDtypeStruct((N, DYN), jnp.float32)
    )(x, W_up, P)


def _step(h, acc2, g_prev, P, b_prev, W1, s_cur, dset, nset,
          mask_prev, relu, pi_in=None, pi_mask=None, emit_pi=False,
          pi_self_mask=None):
    n_out = 3 if emit_pi else 2
    has_pi = pi_in is not None

    def body(*refs):
        (h_ref, acc_ref, g_ref, p_ref, b_ref) = refs[:5]
        i = 5
        if has_pi:
            pi_ref = refs[i]; i += 1
        (w_ref, s_ref) = refs[i:i + 2]
        i += 2
        ho_ref, go_ref = refs[i:i + 2]
        po_ref = refs[i + 2] if emit_pi else None

        P = p_ref[...]
        lay = _col(P, 6)
        conv = _col(P, dset) * (acc_ref[0:N, :] + acc_ref[NSEG:NSEG + N, :]
                                + g_ref[...]) + b_ref[...]
        if mask_prev is None:
            h = conv
        else:
            h = jnp.where(lay == float(mask_prev), conv, h_ref[...])
        if emit_pi:
            po_ref[...] = conv
        if relu:
            h = jnp.maximum(h, 0.0)
        if pi_self_mask is not None:
            h = jnp.where(lay == float(pi_self_mask), conv, h)
        if has_pi:
            h = jnp.where(lay == float(pi_mask), pi_ref[...], h)
        ho_ref[...] = h
        go_ref[...] = (jnp.dot(h, w_ref[...],
                               preferred_element_type=jnp.float32)
                       + s_ref[...]) * _col(P, nset)

    outs = [jax.ShapeDtypeStruct((N, DYN), jnp.float32)] * n_out
    args = [h, acc2, g_prev, P, b_prev]
    if has_pi:
        args.append(pi_in)
    args += [W1, s_cur]
    return pl.pallas_call(body, out_shape=tuple(outs))(*args)


def _final(h, acc2, g_prev, P, b_prev, batch_row, W_lin, b_lin):
    def body(h_ref, acc_ref, g_ref, p_ref, b_ref, bv_ref, wl_ref,
             bl_ref, out_ref):
        P = p_ref[...]
        conv = _col(P, 1) * (acc_ref[0:N, :] + acc_ref[NSEG:NSEG + N, :]
                             + g_ref[...]) + b_ref[...]
        h = jnp.where(_col(P, 6) == 0.0, conv, h_ref[...])
        h = jnp.maximum(h, 0.0)
        t = jnp.dot(h, wl_ref[...], preferred_element_type=jnp.float32)
        gids = lax.broadcasted_iota(jnp.int32, (64, N), 0)
        onehot = (gids == bv_ref[...]).astype(jnp.float32)
        out_ref[...] = jnp.dot(onehot, t,
                               preferred_element_type=jnp.float32) + bl_ref[...]

    return pl.pallas_call(
        body, out_shape=jax.ShapeDtypeStruct((64, 1), jnp.float32)
    )(h, acc2, g_prev, P, b_prev, batch_row, W_lin, b_lin)


def kernel(x, edge_index, feature_mtx_static, layers, inner_edges_0,
           inner_edges_1, inner_edges_2, forward_edges_0, forward_edges_1,
           forward_edges_2, backward_edges_0, backward_edges_1,
           backward_edges_2, batch_vec, W_up, b_up, W_in, b_in, W_fw, b_fw,
           W_bw, b_bw, W_lin, b_lin):
    sets = [edge_index, inner_edges_0, inner_edges_1, inner_edges_2,
            forward_edges_0, forward_edges_1]
    padded = [_pad_edges(e) for e in sets]

    ones_rows = jnp.ones((CHUNK, DYN), jnp.float32)
    zeros_row = jnp.zeros((CHUNK, DYN), jnp.float32)

    deg_cols = []
    for s in range(6):
        _, dsts = padded[s]
        d2 = _scatter_rows(ones_rows, None, dsts, zeros_row, const_rows=True)
        dgr = d2.reshape(2, NSEG, DYN)
        deg_cols.append(dgr[0, :N, 0] + dgr[1, :N, 0])
    deg6 = jnp.stack(deg_cols, axis=1)

    s_in, s_fw, dinv6 = _prep_call(feature_mtx_static, W_in, W_fw, deg6)

    P = jnp.concatenate(
        [dinv6, layers.astype(jnp.float32).reshape(N, 1),
         jnp.zeros((N, 1), jnp.float32)], axis=1)

    batch_row = batch_vec.astype(jnp.int32).reshape(1, N)
    b_up2 = b_up.reshape(1, DYN)
    b_in2 = b_in.reshape(1, DYN)
    b_fw2 = b_fw.reshape(1, DYN)
    Wi1 = W_in[:DYN, :]
    Wf1 = W_fw[:DYN, :]

    def scat(set_id, g):
        src, dst = padded[set_id]
        return _scatter_rows(g, src, dst, zeros_row)

    g = _first_g(x, W_up, P)
    acc = scat(0, g)
    h = g

    pi = None
    for p in range(2):
        if p == 0:
            h, g = _step(h, acc, g, P, b_up2, Wi1, s_in, 0, 1,
                         mask_prev=None, relu=False)
        else:
            h, g = _step(h, acc, g, P, b_in2, Wi1, s_in, 1, 1,
                         mask_prev=0, relu=True)
        acc = scat(1, g)
        h, g = _step(h, acc, g, P, b_in2, Wf1, s_fw, 1, 4,
                     mask_prev=0, relu=False)
        acc = scat(4, g)
        h, g = _step(h, acc, g, P, b_fw2, Wi1, s_in, 4, 2,
                     mask_prev=1, relu=False)
        acc = scat(2, g)
        h, g = _step(h, acc, g, P, b_in2, Wf1, s_fw, 2, 5,
                     mask_prev=1, relu=False)
        acc = scat(5, g)
        h, g = _step(h, acc, g, P, b_fw2, Wi1, s_in, 5, 3,
                     mask_prev=2, relu=False)
        acc = scat(3, g)
        h, g, pi = _step(h, acc, g, P, b_in2, Wi1, s_in, 3, 3,
                         mask_prev=2, relu=True, emit_pi=True, pi_self_mask=1)
        acc = scat(3, g)
        h, g = _step(h, acc, g, P, b_in2, Wi1, s_in, 3, 2,
                     mask_prev=2, relu=False, pi_in=pi, pi_mask=0)
        acc = scat(2, g)
        h, g = _step(h, acc, g, P, b_in2, Wi1, s_in, 2, 1,
                     mask_prev=1, relu=False)
        acc = scat(1, g)

    return _final(h, acc, g, P, b_in2, batch_row, W_lin, b_lin)
